# baseline (speedup 1.0000x reference)
import jax
import jax.numpy as jnp
from jax.experimental import pallas as pl
from jax.experimental.pallas import tpu as pltpu

O_BLK = 1024
K_BLK = 2048


def _body(x_ref, q_ref, lut_ref, o_ref):
    wi = pl.program_id(1)

    @pl.when(wi == 0)
    def _():
        o_ref[...] = jnp.zeros_like(o_ref)

    idx = jnp.right_shift(q_ref[...].astype(jnp.int32), 4) & 0xF
    wgt = jnp.take_along_axis(lut_ref[...], idx, axis=1).astype(jnp.bfloat16)
    o_ref[...] += jax.lax.dot_general(
        x_ref[...], wgt, (((1,), (1,)), ((), ())), preferred_element_type=jnp.float32
    )


def kernel(x, qweight, lut):
    B, S, K = x.shape
    O = qweight.shape[0]
    xb = x.reshape(S, K).astype(jnp.bfloat16)
    q8 = jax.lax.bitcast_convert_type(qweight, jnp.int8).reshape(O, K)

    out = pl.pallas_call(
        _body,
        grid=(O // O_BLK, K // K_BLK),
        in_specs=[
            pl.BlockSpec((S, K_BLK), lambda o, w: (0, w)),
            pl.BlockSpec((O_BLK, K_BLK), lambda o, w: (o, w)),
            pl.BlockSpec((O_BLK, 16), lambda o, w: (o, 0)),
        ],
        out_specs=pl.BlockSpec((S, O_BLK), lambda o, w: (0, o)),
        out_shape=jax.ShapeDtypeStruct((S, O), jnp.float32),
        compiler_params=pltpu.CompilerParams(
            dimension_semantics=("parallel", "arbitrary"),
            vmem_limit_bytes=61 * 1024 * 1024,
        ),
        name="anyprec_linear",
    )(xb, q8, lut)
    return out.reshape(B, S, O)


# speedup vs baseline: 2.2902x; 2.2902x over previous
import jax
import jax.numpy as jnp
from jax.experimental import pallas as pl
from jax.experimental.pallas import tpu as pltpu

O_BLK = 1024
W_BLK = 512


def _body(x0_ref, x1_ref, x2_ref, x3_ref, q_ref, lut_ref, o_ref):
    wi = pl.program_id(1)

    @pl.when(wi == 0)
    def _():
        o_ref[...] = jnp.zeros_like(o_ref)

    q = q_ref[...]
    lut = lut_ref[...]
    planes = []
    for b in range(4):
        idx = jnp.right_shift(q, 8 * b + 4) & 0xF
        planes.append(jnp.take_along_axis(lut, idx, axis=1).astype(jnp.bfloat16))
    wcat = jnp.concatenate(planes, axis=1)
    xcat = jnp.concatenate(
        [x0_ref[...], x1_ref[...], x2_ref[...], x3_ref[...]], axis=1
    )
    o_ref[...] += jax.lax.dot_general(
        xcat, wcat, (((1,), (1,)), ((), ())), preferred_element_type=jnp.float32
    )


def kernel(x, qweight, lut):
    B, S, K = x.shape
    O = qweight.shape[0]
    NW = K // 4
    NWB = NW // W_BLK
    xp = x.astype(jnp.bfloat16).reshape(S, NW, 4).transpose(0, 2, 1).reshape(S, K)

    def x_spec(b):
        return pl.BlockSpec((S, W_BLK), lambda o, w, b=b: (0, b * NWB + w))

    out = pl.pallas_call(
        _body,
        grid=(O // O_BLK, NWB),
        in_specs=[
            x_spec(0),
            x_spec(1),
            x_spec(2),
            x_spec(3),
            pl.BlockSpec((O_BLK, W_BLK), lambda o, w: (o, w)),
            pl.BlockSpec((O_BLK, 16), lambda o, w: (o, 0)),
        ],
        out_specs=pl.BlockSpec((S, O_BLK), lambda o, w: (0, o)),
        out_shape=jax.ShapeDtypeStruct((S, O), jnp.float32),
        compiler_params=pltpu.CompilerParams(
            dimension_semantics=("parallel", "arbitrary"),
            vmem_limit_bytes=61 * 1024 * 1024,
        ),
        name="anyprec_linear",
    )(xp, xp, xp, xp, qweight, lut)
    return out.reshape(B, S, O)
